# baseline (speedup 1.0000x reference)
import jax
import jax.numpy as jnp
from jax.experimental import pallas as pl


_BLOCK_C = 32000


def _broadcast_body(w_ref, o_ref):
    o_ref[...] = jnp.broadcast_to(w_ref[...], o_ref.shape)


def kernel(edge_index, W):
    E = edge_index.shape[1]
    D = W.shape[1]
    w_col = W.reshape(D, 1)
    out_t = pl.pallas_call(
        _broadcast_body,
        grid=(E // _BLOCK_C,),
        in_specs=[pl.BlockSpec((D, 1), lambda i: (0, 0))],
        out_specs=pl.BlockSpec((D, _BLOCK_C), lambda i: (0, i)),
        out_shape=jax.ShapeDtypeStruct((D, E), jnp.float32),
    )(w_col)
    return out_t.T
